# baseline (speedup 1.0000x reference)
import functools

import jax
import jax.numpy as jnp
from jax.experimental import pallas as pl
from jax.experimental.pallas import tpu as pltpu
from jax.experimental.pallas import tpu_sc as plsc

_F32 = jnp.float32
_BF16 = jnp.bfloat16


def _bdot(a, b):
    return jnp.dot(a.astype(_BF16), b.astype(_BF16),
                   preferred_element_type=_F32)


def _row_dot(vrow, m):
    return jax.lax.dot_general(
        vrow.astype(_BF16), m.astype(_BF16),
        (((1,), (1,)), ((), ())), preferred_element_type=_F32)


def _ln(h, g, b):
    m = jnp.mean(h, axis=-1, keepdims=True)
    d = h - m
    v = jnp.mean(d * d, axis=-1, keepdims=True)
    return d * jax.lax.rsqrt(v + 1e-5) * g + b


def _shift_dn(x):
    z = jnp.zeros((1, x.shape[1]), x.dtype)
    return jnp.concatenate([z, x[:-1, :]], axis=0)


def _shift_up(x):
    z = jnp.zeros((1, x.shape[1]), x.dtype)
    return jnp.concatenate([x[1:, :], z], axis=0)


def _conv3(h, wwide, b, C):
    O = _bdot(h, wwide)
    return (_shift_dn(O[:, :C]) + O[:, C:2 * C]
            + _shift_up(O[:, 2 * C:]) + b)


def _vp_body(h, C, w1, b1, g1, be1, w2, b2, g2, be2, lwrow):
    h = _ln(jax.nn.relu(_conv3(h, w1, b1, C)), g1, be1)
    h = _ln(jax.nn.relu(_conv3(h, w2, b2, C)), g2, be2)
    return _row_dot(lwrow, h)


def _idx_kernel(S, T, SENT, durc_ref, maxd_ref, gidx_ref, maskf_ref):
    b = pl.program_id(0)
    durcol = durc_ref[0]
    r = jax.lax.broadcasted_iota(jnp.int32, (S, S), 0)
    c = jax.lax.broadcasted_iota(jnp.int32, (S, S), 1)
    lower = (r >= c).astype(_BF16)
    cumcol = _bdot(lower, durcol)
    mel_len = cumcol[S - 1, 0]
    lim = jnp.minimum(mel_len, maxd_ref[0, 0])

    trow = jax.lax.broadcasted_iota(jnp.int32, (1, T), 1).astype(_F32)
    M = (cumcol <= trow).astype(_BF16)
    idxrow = _bdot(jnp.full((1, S), 1.0, _F32), M)
    validrow = trow < lim
    offs = (b * S).astype(_F32)
    gidxrow = jnp.where(validrow, idxrow + offs, float(SENT))
    gidx_ref[0, 0, :] = gidxrow[0].astype(jnp.int32)
    maskf_ref[0, 0, :] = 1.0 - validrow[0].astype(_F32)


def _sc_gather(xpad, gidx2d):
    BT = gidx2d.shape[1]
    D = xpad.shape[1]
    W = 128

    mesh = plsc.VectorSubcoreMesh(core_axis_name="core",
                                  subcore_axis_name="subcore")

    @pl.kernel(out_type=jax.ShapeDtypeStruct((BT, D), xpad.dtype), mesh=mesh)
    def sckern(x_hbm, i_hbm, o_hbm):
        def body(i_vmem, o_vmem):
            pltpu.sync_copy(x_hbm.at[i_vmem.at[0]], o_vmem)

        pltpu.emit_pipeline(
            body,
            grid=(BT // W,),
            in_specs=[pl.BlockSpec((1, W), lambda i: (0, i))],
            out_specs=[pl.BlockSpec((W, D), lambda i: (i, 0))],
            core_axis_name=("core", "subcore"),
            dimension_semantics=(pltpu.PARALLEL,),
        )(i_hbm, o_hbm)

    return sckern(xpad, gidx2d)


def _dur_kernel(C, x_ref, dw1, db1, dg1, dbe1, dw2, db2, dg2, dbe2, dlw,
                lbs_ref, dpred_ref):
    dpred = _vp_body(x_ref[0], C, dw1[...], db1[...], dg1[...], dbe1[...],
                     dw2[...], db2[...], dg2[...], dbe2[...], dlw[...])
    dpred_ref[0, 0, :] = dpred[0] + lbs_ref[0, 0]


def _pe_kernel(T, D, C, NB,
               xe0_ref, ptrg_ref, lob_ref, hib_ref, maskf_ref,
               pw1, pb1, pg1, pbe1, pw2, pb2, pg2, pbe2, plw,
               ew1, eb1, eg1, ebe1, ew2, eb2, eg2, ebe2, elw,
               tabs_ref, lbs_ref,
               xe_ref, pemb_ref, eemb_ref, ppred_ref, epred_ref):
    xe0 = xe0_ref[0]
    ptrg = ptrg_ref[0]
    plb = lbs_ref[0, 1]
    elb = lbs_ref[0, 2]
    validrow = 1.0 - maskf_ref[0]

    pv = ptrg.reshape(T, 1)
    onehot = ((lob_ref[0:1, :] < pv) & (pv <= hib_ref[0:1, :])).astype(_BF16)
    embs = jnp.dot(onehot, tabs_ref[...].astype(_BF16),
                   preferred_element_type=_F32)
    pemb = embs[:, :D]
    eemb = embs[:, D:]
    pemb_ref[0] = pemb
    eemb_ref[0] = eemb

    ppred = _vp_body(xe0, C, pw1[...], pb1[...], pg1[...], pbe1[...],
                     pw2[...], pb2[...], pg2[...], pbe2[...], plw[...])
    ppred_ref[0, 0, :] = (ppred[0] + plb) * validrow[0]
    xe1 = xe0 + pemb

    epred = _vp_body(xe1, C, ew1[...], eb1[...], eg1[...], ebe1[...],
                     ew2[...], eb2[...], eg2[...], ebe2[...], elw[...])
    epred_ref[0, 0, :] = (epred[0] + elb) * validrow[0]
    xe_ref[0] = xe1 + eemb


def _wide_conv_w(w):
    K = w.shape[2]
    return jnp.concatenate([w[:, :, k].T for k in range(K)], axis=1)


def _vp_args(p):
    C = p['c1b'].shape[0]
    return (
        _wide_conv_w(p['c1w']).astype(_BF16), p['c1b'].reshape(1, C),
        p['g1'].reshape(1, C), p['b1'].reshape(1, C),
        _wide_conv_w(p['c2w']).astype(_BF16), p['c2b'].reshape(1, C),
        p['g2'].reshape(1, C), p['b2'].reshape(1, C),
        p['lw'].reshape(1, C).astype(_BF16),
    )


def kernel(x, dur_trg, pitch_trg, energy_trg, src_mask, max_dur,
           dp, pp, ep, pitch_bins, energy_bins, pitch_table, energy_table):
    B, S, D = x.shape
    T = pitch_trg.shape[1]
    C = dp['c1b'].shape[0]
    NB = pitch_table.shape[0]
    SENT = B * S

    durc = dur_trg.astype(_F32).reshape(B, S, 1)
    ptrg = pitch_trg.reshape(B, 1, T)
    binsf = pitch_bins.astype(_F32)
    lob = jnp.concatenate([jnp.full((1,), -1e30, _F32), binsf]).reshape(1, NB)
    hib = jnp.concatenate([binsf, jnp.full((1,), 1e30, _F32)]).reshape(1, NB)
    lob = jnp.broadcast_to(lob, (8, NB))
    hib = jnp.broadcast_to(hib, (8, NB))
    maxd_arr = jnp.full((8, 128), max_dur, _F32)
    tabs = jnp.concatenate([pitch_table, energy_table], axis=1)
    lbs = jnp.stack([dp['lb'], pp['lb'], ep['lb']]).reshape(1, 3)
    lbs = jnp.broadcast_to(jnp.pad(lbs, ((0, 0), (0, 125))), (8, 128))

    def full(a):
        return pl.BlockSpec(a.shape, lambda b: (0,) * a.ndim)

    row_spec = pl.BlockSpec((1, 1, T), lambda b: (b, 0, 0))

    gidx, maskf = pl.pallas_call(
        functools.partial(_idx_kernel, S, T, SENT),
        grid=(B,),
        in_specs=[pl.BlockSpec((1, S, 1), lambda b: (b, 0, 0)),
                  full(maxd_arr)],
        out_specs=[row_spec, row_spec],
        out_shape=[jax.ShapeDtypeStruct((B, 1, T), jnp.int32),
                   jax.ShapeDtypeStruct((B, 1, T), _F32)],
    )(durc, maxd_arr)

    xpad = jnp.concatenate([x.reshape(B * S, D), jnp.zeros((8, D), _F32)])
    xe0 = _sc_gather(xpad, gidx.reshape(1, B * T))
    xe0 = xe0.reshape(B, T, D)

    dvp = _vp_args(dp)
    dpred = pl.pallas_call(
        functools.partial(_dur_kernel, C),
        grid=(B,),
        in_specs=[pl.BlockSpec((1, S, D), lambda b: (b, 0, 0))]
        + [full(a) for a in dvp] + [full(lbs)],
        out_specs=[pl.BlockSpec((1, 1, S), lambda b: (b, 0, 0))],
        out_shape=[jax.ShapeDtypeStruct((B, 1, S), _F32)],
    )(x, *dvp, lbs)[0]

    pvp = _vp_args(pp)
    evp = _vp_args(ep)
    xe, pemb, eemb, ppred, epred = pl.pallas_call(
        functools.partial(_pe_kernel, T, D, C, NB),
        grid=(B,),
        in_specs=[pl.BlockSpec((1, T, D), lambda b: (b, 0, 0)),
                  row_spec, full(lob), full(hib), row_spec]
        + [full(a) for a in pvp] + [full(a) for a in evp]
        + [full(tabs), full(lbs)],
        out_specs=[pl.BlockSpec((1, T, D), lambda b: (b, 0, 0)),
                   pl.BlockSpec((1, T, D), lambda b: (b, 0, 0)),
                   pl.BlockSpec((1, T, D), lambda b: (b, 0, 0)),
                   row_spec, row_spec],
        out_shape=[jax.ShapeDtypeStruct((B, T, D), _F32),
                   jax.ShapeDtypeStruct((B, T, D), _F32),
                   jax.ShapeDtypeStruct((B, T, D), _F32),
                   jax.ShapeDtypeStruct((B, 1, T), _F32),
                   jax.ShapeDtypeStruct((B, 1, T), _F32)],
    )(xe0, ptrg, lob, hib, maskf, *pvp, *evp, tabs, lbs)

    mel_mask = maskf.reshape(B, T) > 0.5
    log_dur_pred = jnp.where(src_mask, 0.0, dpred.reshape(B, S))
    return (xe, mel_mask, log_dur_pred, dur_trg,
            ppred.reshape(B, T), pemb, epred.reshape(B, T), eemb)


# speedup vs baseline: 1.5955x; 1.5955x over previous
import functools

import jax
import jax.numpy as jnp
from jax.experimental import pallas as pl
from jax.experimental.pallas import tpu as pltpu
from jax.experimental.pallas import tpu_sc as plsc

_F32 = jnp.float32
_BF16 = jnp.bfloat16


def _bdot(a, b):
    return jnp.dot(a.astype(_BF16), b.astype(_BF16),
                   preferred_element_type=_F32)


def _row_dot(vrow, m):
    return jax.lax.dot_general(
        vrow.astype(_BF16), m.astype(_BF16),
        (((1,), (1,)), ((), ())), preferred_element_type=_F32)


def _ln(h, g, b):
    m = jnp.mean(h, axis=-1, keepdims=True)
    d = h - m
    v = jnp.mean(d * d, axis=-1, keepdims=True)
    return d * jax.lax.rsqrt(v + 1e-5) * g + b


def _shift_dn(x):
    z = jnp.zeros((1, x.shape[1]), x.dtype)
    return jnp.concatenate([z, x[:-1, :]], axis=0)


def _shift_up(x):
    z = jnp.zeros((1, x.shape[1]), x.dtype)
    return jnp.concatenate([x[1:, :], z], axis=0)


def _conv3(h, wwide, b, C):
    O = _bdot(h, wwide)
    return (_shift_dn(O[:, :C]) + O[:, C:2 * C]
            + _shift_up(O[:, 2 * C:]) + b)


def _vp_body(h, C, w1, b1, g1, be1, w2, b2, g2, be2, lwrow):
    h = _ln(jax.nn.relu(_conv3(h, w1, b1, C)), g1, be1)
    h = _ln(jax.nn.relu(_conv3(h, w2, b2, C)), g2, be2)
    return _row_dot(lwrow, h)


def _bidx_kernel(T, NB, ptrg_ref, binsc_ref, pidx_ref):
    prow = ptrg_ref[0]
    binscol = binsc_ref[:, 0:1]
    M = (binscol < prow).astype(_BF16)
    idxrow = _bdot(jnp.full((1, NB), 1.0, _F32), M)
    pidx_ref[0, 0, :] = idxrow[0].astype(jnp.int32)


def _sc_gather(ptab, etab, pidx2d):
    BT = pidx2d.shape[1]
    D = ptab.shape[1]
    W = 128

    mesh = plsc.VectorSubcoreMesh(core_axis_name="core",
                                  subcore_axis_name="subcore")

    @pl.kernel(out_type=[jax.ShapeDtypeStruct((BT, D), ptab.dtype),
                         jax.ShapeDtypeStruct((BT, D), etab.dtype)],
               mesh=mesh)
    def sckern(p_hbm, e_hbm, i_hbm, op_hbm, oe_hbm):
        def gather_into(t_hbm, o_hbm):
            def body(i_vmem, o_vmem):
                pltpu.sync_copy(t_hbm.at[i_vmem.at[0]], o_vmem)

            pltpu.emit_pipeline(
                body,
                grid=(BT // W,),
                in_specs=[pl.BlockSpec((1, W), lambda i: (0, i))],
                out_specs=[pl.BlockSpec((W, D), lambda i: (i, 0))],
                core_axis_name=("core", "subcore"),
                dimension_semantics=(pltpu.PARALLEL,),
            )(i_hbm, o_hbm)

        gather_into(p_hbm, op_hbm)
        gather_into(e_hbm, oe_hbm)

    return sckern(ptab, etab, pidx2d)


def _mega_kernel(S, T, D, C, NB,
                 x_ref, durc_ref, ptrg_ref, lob_ref, hib_ref, maxd_ref,
                 dw1, db1, dg1, dbe1, dw2, db2, dg2, dbe2, dlw,
                 pw1, pb1, pg1, pbe1, pw2, pb2, pg2, pbe2, plw,
                 ew1, eb1, eg1, ebe1, ew2, eb2, eg2, ebe2, elw,
                 tabs_ref, lbs_ref,
                 xe_ref, ppred_ref, epred_ref, dpred_ref, maskf_ref):
    xb = x_ref[0]
    durcol = durc_ref[0]
    ptrg = ptrg_ref[0]
    maxd = maxd_ref[0, 0]
    dlb = lbs_ref[0, 0]
    plb = lbs_ref[0, 1]
    elb = lbs_ref[0, 2]

    dpred = _vp_body(xb, C, dw1[...], db1[...], dg1[...], dbe1[...],
                     dw2[...], db2[...], dg2[...], dbe2[...], dlw[...])
    dpred_ref[0, 0, :] = dpred[0] + dlb

    r = jax.lax.broadcasted_iota(jnp.int32, (S, S), 0)
    c = jax.lax.broadcasted_iota(jnp.int32, (S, S), 1)
    lower = (r >= c).astype(_BF16)
    cumcol = _bdot(lower, durcol)
    cum = jnp.transpose(cumcol)
    cum_prev = cum - jnp.transpose(durcol)
    mel_len = cumcol[S - 1, 0]
    lim = jnp.minimum(mel_len, maxd)
    cumc = jnp.minimum(cum, lim)

    tt = jax.lax.broadcasted_iota(jnp.int32, (T, 1), 0).astype(_F32)
    E = ((cum_prev <= tt) & (tt < cumc)).astype(_BF16)
    xe0 = jnp.dot(E, xb.astype(_BF16), preferred_element_type=_F32)
    trow = jax.lax.broadcasted_iota(jnp.int32, (1, T), 1).astype(_F32)
    maskf_ref[0, 0, :] = (trow[0] >= lim).astype(_F32)
    validrow = (trow < lim).astype(_F32)

    pv = ptrg.reshape(T, 1)
    onehot = ((lob_ref[0:1, :] < pv) & (pv <= hib_ref[0:1, :])).astype(_BF16)
    embs = jnp.dot(onehot, tabs_ref[...].astype(_BF16),
                   preferred_element_type=_F32)
    pemb = embs[:, :D]
    eemb = embs[:, D:]

    ppred = _vp_body(xe0, C, pw1[...], pb1[...], pg1[...], pbe1[...],
                     pw2[...], pb2[...], pg2[...], pbe2[...], plw[...])
    ppred_ref[0, 0, :] = (ppred[0] + plb) * validrow[0]
    xe1 = xe0 + pemb

    epred = _vp_body(xe1, C, ew1[...], eb1[...], eg1[...], ebe1[...],
                     ew2[...], eb2[...], eg2[...], ebe2[...], elw[...])
    epred_ref[0, 0, :] = (epred[0] + elb) * validrow[0]
    xe_ref[0] = xe1 + eemb


def _wide_conv_w(w):
    K = w.shape[2]
    return jnp.concatenate([w[:, :, k].T for k in range(K)], axis=1)


def _vp_args(p):
    C = p['c1b'].shape[0]
    return (
        _wide_conv_w(p['c1w']).astype(_BF16), p['c1b'].reshape(1, C),
        p['g1'].reshape(1, C), p['b1'].reshape(1, C),
        _wide_conv_w(p['c2w']).astype(_BF16), p['c2b'].reshape(1, C),
        p['g2'].reshape(1, C), p['b2'].reshape(1, C),
        p['lw'].reshape(1, C).astype(_BF16),
    )


def kernel(x, dur_trg, pitch_trg, energy_trg, src_mask, max_dur,
           dp, pp, ep, pitch_bins, energy_bins, pitch_table, energy_table):
    B, S, D = x.shape
    T = pitch_trg.shape[1]
    C = dp['c1b'].shape[0]
    NB = pitch_table.shape[0]

    durc = dur_trg.astype(_F32).reshape(B, S, 1)
    ptrg = pitch_trg.reshape(B, 1, T)
    binsf = pitch_bins.astype(_F32)
    lob = jnp.concatenate([jnp.full((1,), -1e30, _F32), binsf]).reshape(1, NB)
    hib = jnp.concatenate([binsf, jnp.full((1,), 1e30, _F32)]).reshape(1, NB)
    lob = jnp.broadcast_to(lob, (8, NB))
    hib = jnp.broadcast_to(hib, (8, NB))
    binscol = jnp.broadcast_to(
        jnp.concatenate([binsf, jnp.full((1,), 1e30, _F32)]).reshape(NB, 1),
        (NB, 128))
    maxd_arr = jnp.full((8, 128), max_dur, _F32)
    tabs = jnp.concatenate([pitch_table, energy_table], axis=1)
    lbs = jnp.stack([dp['lb'], pp['lb'], ep['lb']]).reshape(1, 3)
    lbs = jnp.broadcast_to(jnp.pad(lbs, ((0, 0), (0, 125))), (8, 128))

    def full(a):
        return pl.BlockSpec(a.shape, lambda b: (0,) * a.ndim)

    row_spec = pl.BlockSpec((1, 1, T), lambda b: (b, 0, 0))

    pidx = pl.pallas_call(
        functools.partial(_bidx_kernel, T, NB),
        grid=(B,),
        in_specs=[row_spec, full(binscol)],
        out_specs=[row_spec],
        out_shape=[jax.ShapeDtypeStruct((B, 1, T), jnp.int32)],
    )(ptrg, binscol)[0]

    pemb, eemb = _sc_gather(pitch_table, energy_table, pidx.reshape(1, B * T))
    pemb = pemb.reshape(B, T, D)
    eemb = eemb.reshape(B, T, D)

    vp_all = _vp_args(dp) + _vp_args(pp) + _vp_args(ep)
    xe, ppred, epred, dpred, maskf = pl.pallas_call(
        functools.partial(_mega_kernel, S, T, D, C, NB),
        grid=(B,),
        in_specs=[pl.BlockSpec((1, S, D), lambda b: (b, 0, 0)),
                  pl.BlockSpec((1, S, 1), lambda b: (b, 0, 0)),
                  row_spec, full(lob), full(hib), full(maxd_arr)]
        + [full(a) for a in vp_all]
        + [full(tabs), full(lbs)],
        out_specs=[pl.BlockSpec((1, T, D), lambda b: (b, 0, 0)),
                   row_spec, row_spec,
                   pl.BlockSpec((1, 1, S), lambda b: (b, 0, 0)),
                   row_spec],
        out_shape=[jax.ShapeDtypeStruct((B, T, D), _F32),
                   jax.ShapeDtypeStruct((B, 1, T), _F32),
                   jax.ShapeDtypeStruct((B, 1, T), _F32),
                   jax.ShapeDtypeStruct((B, 1, S), _F32),
                   jax.ShapeDtypeStruct((B, 1, T), _F32)],
    )(x, durc, ptrg, lob, hib, maxd_arr, *vp_all, tabs, lbs)

    mel_mask = maskf.reshape(B, T) > 0.5
    log_dur_pred = jnp.where(src_mask, 0.0, dpred.reshape(B, S))
    return (xe, mel_mask, log_dur_pred, dur_trg,
            ppred.reshape(B, T), pemb, epred.reshape(B, T), eemb)


# speedup vs baseline: 1.6581x; 1.0393x over previous
import functools

import jax
import jax.numpy as jnp
from jax.experimental import pallas as pl
from jax.experimental.pallas import tpu as pltpu
from jax.experimental.pallas import tpu_sc as plsc

_F32 = jnp.float32
_BF16 = jnp.bfloat16


def _bdot(a, b):
    return jnp.dot(a.astype(_BF16), b.astype(_BF16),
                   preferred_element_type=_F32)


def _tdot(a, bt):
    return jax.lax.dot_general(
        a.astype(_BF16), bt.astype(_BF16),
        (((1,), (1,)), ((), ())), preferred_element_type=_F32)


def _ln(h, g, b):
    m = jnp.mean(h, axis=-1, keepdims=True)
    d = h - m
    v = jnp.mean(d * d, axis=-1, keepdims=True)
    return d * jax.lax.rsqrt(v + 1e-5) * g + b


def _shift_dn(x):
    z = jnp.zeros((1, x.shape[1]), x.dtype)
    return jnp.concatenate([z, x[:-1, :]], axis=0)


def _shift_up(x):
    z = jnp.zeros((1, x.shape[1]), x.dtype)
    return jnp.concatenate([x[1:, :], z], axis=0)


def _conv3(h, w_ref, b):
    h16 = h.astype(_BF16)
    return (_shift_dn(_tdot(h16, w_ref[0])) + _tdot(h16, w_ref[1])
            + _shift_up(_tdot(h16, w_ref[2])) + b)


def _vp_body(h, w1, b1, g1, be1, w2, b2, g2, be2, lwrow):
    h = _ln(jax.nn.relu(_conv3(h, w1, b1)), g1, be1)
    h = _ln(jax.nn.relu(_conv3(h, w2, b2)), g2, be2)
    return jax.lax.dot_general(
        lwrow.astype(_BF16), h.astype(_BF16),
        (((1,), (1,)), ((), ())), preferred_element_type=_F32)


def _bidx_kernel(NB, ptrg_ref, binsc_ref, pidx_ref):
    prow = ptrg_ref[0]
    binscol = binsc_ref[:, 0:1]
    M = (binscol < prow).astype(_BF16)
    idxrow = _bdot(jnp.full((1, NB), 1.0, _F32), M)
    pidx_ref[0, 0, :] = idxrow[0].astype(jnp.int32)


def _sc_gather(ptab, etab, pidx2d):
    BT = pidx2d.shape[1]
    D = ptab.shape[1]
    W = 128

    mesh = plsc.VectorSubcoreMesh(core_axis_name="core",
                                  subcore_axis_name="subcore")

    @pl.kernel(out_type=[jax.ShapeDtypeStruct((BT, D), ptab.dtype),
                         jax.ShapeDtypeStruct((BT, D), etab.dtype)],
               mesh=mesh)
    def sckern(p_hbm, e_hbm, i_hbm, op_hbm, oe_hbm):
        def gather_into(t_hbm, o_hbm):
            def body(i_vmem, o_vmem):
                pltpu.sync_copy(t_hbm.at[i_vmem.at[0]], o_vmem)

            pltpu.emit_pipeline(
                body,
                grid=(BT // W,),
                in_specs=[pl.BlockSpec((1, W), lambda i: (0, i))],
                out_specs=[pl.BlockSpec((W, D), lambda i: (i, 0))],
                core_axis_name=("core", "subcore"),
                dimension_semantics=(pltpu.PARALLEL,),
            )(i_hbm, o_hbm)

        gather_into(p_hbm, op_hbm)
        gather_into(e_hbm, oe_hbm)

    return sckern(ptab, etab, pidx2d)


def _mega_kernel(S, T, D, C, NB,
                 x_ref, durc_ref, ptrg_ref, hib_ref, maxd_ref,
                 dw1, db1, dg1, dbe1, dw2, db2, dg2, dbe2, dlw,
                 pw1, pb1, pg1, pbe1, pw2, pb2, pg2, pbe2, plw,
                 ew1, eb1, eg1, ebe1, ew2, eb2, eg2, ebe2, elw,
                 tabs_ref,
                 xe_ref, ppred_ref, epred_ref, dpred_ref, maskf_ref):
    xb = x_ref[0]
    durcol = durc_ref[0]
    ptrg = ptrg_ref[0]
    maxd = maxd_ref[0, 0]

    dpred = _vp_body(xb, dw1, db1[...], dg1[...], dbe1[...],
                     dw2, db2[...], dg2[...], dbe2[...], dlw[...])
    dpred_ref[0, 0, :] = dpred[0]

    r = jax.lax.broadcasted_iota(jnp.int32, (S, S), 0)
    c = jax.lax.broadcasted_iota(jnp.int32, (S, S), 1)
    upper = (r <= c).astype(_BF16)
    cum = jax.lax.dot_general(
        durcol.astype(_BF16), upper, (((0,), (0,)), ((), ())),
        preferred_element_type=_F32)
    durrow = jax.lax.dot_general(
        durcol.astype(_BF16), (r == c).astype(_BF16), (((0,), (0,)), ((), ())),
        preferred_element_type=_F32)
    cum_prev = cum - durrow
    mel_len = cum[0, S - 1]
    lim = jnp.minimum(mel_len, maxd)
    cumc = jnp.minimum(cum, lim)

    tt = jax.lax.broadcasted_iota(jnp.int32, (T, 1), 0).astype(_F32)
    E = ((cum_prev <= tt) & (tt < cumc)).astype(_BF16)
    xe0 = jnp.dot(E, xb.astype(_BF16), preferred_element_type=_F32)
    trow = jax.lax.broadcasted_iota(jnp.int32, (1, T), 1).astype(_F32)
    maskf_ref[0, 0, :] = (trow[0] >= lim).astype(_F32)

    pv = ptrg.reshape(T, 1)
    hib = hib_ref[0:1, :]
    lob = jnp.concatenate([jnp.full((1, 1), -1e30, _F32), hib[:, :NB - 1]],
                          axis=1)
    onehot = ((lob < pv) & (pv <= hib)).astype(_BF16)
    embs = jnp.dot(onehot, tabs_ref[...].astype(_BF16),
                   preferred_element_type=_F32)
    pemb = embs[:, :D]
    eemb = embs[:, D:]

    ppred = _vp_body(xe0, pw1, pb1[...], pg1[...], pbe1[...],
                     pw2, pb2[...], pg2[...], pbe2[...], plw[...])
    ppred_ref[0, 0, :] = ppred[0]
    xe1 = xe0 + pemb

    epred = _vp_body(xe1, ew1, eb1[...], eg1[...], ebe1[...],
                     ew2, eb2[...], eg2[...], ebe2[...], elw[...])
    epred_ref[0, 0, :] = epred[0]
    xe_ref[0] = xe1 + eemb


def _vp_args(p):
    C = p['c1b'].shape[0]
    return (
        p['c1w'].transpose(2, 0, 1).astype(_BF16), p['c1b'].reshape(1, C),
        p['g1'].reshape(1, C), p['b1'].reshape(1, C),
        p['c2w'].transpose(2, 0, 1).astype(_BF16), p['c2b'].reshape(1, C),
        p['g2'].reshape(1, C), p['b2'].reshape(1, C),
        p['lw'].reshape(1, C).astype(_BF16),
    )


def kernel(x, dur_trg, pitch_trg, energy_trg, src_mask, max_dur,
           dp, pp, ep, pitch_bins, energy_bins, pitch_table, energy_table):
    B, S, D = x.shape
    T = pitch_trg.shape[1]
    C = dp['c1b'].shape[0]
    NB = pitch_table.shape[0]
    G = 4

    durc = dur_trg.astype(_F32).reshape(B, S, 1)
    ptrg = pitch_trg.reshape(B, 1, T)
    binsf = pitch_bins.astype(_F32)
    hib = jnp.concatenate([binsf, jnp.full((1,), 1e30, _F32)]).reshape(1, NB)
    hib8 = jnp.broadcast_to(hib, (8, NB))
    binscol = jnp.broadcast_to(hib.reshape(NB, 1), (NB, 128))
    maxd_arr = jnp.full((8, 128), max_dur, _F32)
    tabs = jnp.concatenate([pitch_table, energy_table], axis=1)

    def full(a):
        return pl.BlockSpec(a.shape, lambda b: (0,) * a.ndim)

    row_spec = pl.BlockSpec((1, 1, T), lambda b: (b, 0, 0))

    ptrg_flat = pitch_trg.reshape(B // G, 1, G * T)
    pidx = pl.pallas_call(
        functools.partial(_bidx_kernel, NB),
        grid=(B // G,),
        in_specs=[pl.BlockSpec((1, 1, G * T), lambda b: (b, 0, 0)),
                  full(binscol)],
        out_specs=[pl.BlockSpec((1, 1, G * T), lambda b: (b, 0, 0))],
        out_shape=[jax.ShapeDtypeStruct((B // G, 1, G * T), jnp.int32)],
    )(ptrg_flat, binscol)[0]

    pemb, eemb = _sc_gather(pitch_table, energy_table, pidx.reshape(1, B * T))
    pemb = pemb.reshape(B, T, D)
    eemb = eemb.reshape(B, T, D)

    vp_all = _vp_args(dp) + _vp_args(pp) + _vp_args(ep)
    xe, ppred, epred, dpred, maskf = pl.pallas_call(
        functools.partial(_mega_kernel, S, T, D, C, NB),
        grid=(B,),
        in_specs=[pl.BlockSpec((1, S, D), lambda b: (b, 0, 0)),
                  pl.BlockSpec((1, S, 1), lambda b: (b, 0, 0)),
                  row_spec, full(hib8), full(maxd_arr)]
        + [full(a) for a in vp_all]
        + [full(tabs)],
        out_specs=[pl.BlockSpec((1, T, D), lambda b: (b, 0, 0)),
                   row_spec, row_spec,
                   pl.BlockSpec((1, 1, S), lambda b: (b, 0, 0)),
                   row_spec],
        out_shape=[jax.ShapeDtypeStruct((B, T, D), _F32),
                   jax.ShapeDtypeStruct((B, 1, T), _F32),
                   jax.ShapeDtypeStruct((B, 1, T), _F32),
                   jax.ShapeDtypeStruct((B, 1, S), _F32),
                   jax.ShapeDtypeStruct((B, 1, T), _F32)],
    )(x, durc, ptrg, hib8, maxd_arr, *vp_all, tabs)

    mel_mask = maskf.reshape(B, T) > 0.5
    validf = 1.0 - maskf.reshape(B, T)
    log_dur_pred = jnp.where(src_mask, 0.0, dpred.reshape(B, S) + dp['lb'])
    pitch_pred = (ppred.reshape(B, T) + pp['lb']) * validf
    energy_pred = (epred.reshape(B, T) + ep['lb']) * validf
    return (xe, mel_mask, log_dur_pred, dur_trg,
            pitch_pred, pemb, energy_pred, eemb)


# speedup vs baseline: 1.7899x; 1.0794x over previous
import functools

import jax
import jax.numpy as jnp
from jax.experimental import pallas as pl
from jax.experimental.pallas import tpu as pltpu
from jax.experimental.pallas import tpu_sc as plsc

_F32 = jnp.float32
_BF16 = jnp.bfloat16


def _bdot(a, b):
    return jnp.dot(a.astype(_BF16), b.astype(_BF16),
                   preferred_element_type=_F32)


def _tdot(a, bt):
    return jax.lax.dot_general(
        a.astype(_BF16), bt.astype(_BF16),
        (((1,), (1,)), ((), ())), preferred_element_type=_F32)


def _ln(h, g, b):
    m = jnp.mean(h, axis=-1, keepdims=True)
    d = h - m
    v = jnp.mean(d * d, axis=-1, keepdims=True)
    return d * jax.lax.rsqrt(v + 1e-5) * g + b


def _shift_dn(x):
    z = jnp.zeros((1, x.shape[1]), x.dtype)
    return jnp.concatenate([z, x[:-1, :]], axis=0)


def _shift_up(x):
    z = jnp.zeros((1, x.shape[1]), x.dtype)
    return jnp.concatenate([x[1:, :], z], axis=0)


def _conv3(h, w_ref, b):
    h16 = h.astype(_BF16)
    w = w_ref[...].astype(_BF16)
    return (_shift_dn(_tdot(h16, w[0])) + _tdot(h16, w[1])
            + _shift_up(_tdot(h16, w[2])) + b)


def _vp_body(h, w1, b1, g1, be1, w2, b2, g2, be2, lwrow):
    h = _ln(jax.nn.relu(_conv3(h, w1, b1)), g1, be1)
    h = _ln(jax.nn.relu(_conv3(h, w2, b2)), g2, be2)
    return jax.lax.dot_general(
        lwrow.astype(_BF16), h.astype(_BF16),
        (((1,), (1,)), ((), ())), preferred_element_type=_F32)


def _bidx_kernel(NB, ptrg_ref, binsc_ref, pidx_ref):
    prow = ptrg_ref[0]
    binscol = binsc_ref[:, 0:1]
    M = (binscol < prow).astype(_BF16)
    idxrow = _bdot(jnp.full((1, NB), 1.0, _F32), M)
    pidx_ref[0, 0, :] = idxrow[0].astype(jnp.int32)


def _sc_gather(ptab, etab, pidx2d):
    BT = pidx2d.shape[1]
    D = ptab.shape[1]
    W = 128

    mesh = plsc.VectorSubcoreMesh(core_axis_name="core",
                                  subcore_axis_name="subcore")

    @pl.kernel(out_type=[jax.ShapeDtypeStruct((BT, D), ptab.dtype),
                         jax.ShapeDtypeStruct((BT, D), etab.dtype)],
               mesh=mesh)
    def sckern(p_hbm, e_hbm, i_hbm, op_hbm, oe_hbm):
        def gather_into(t_hbm, o_hbm):
            def body(i_vmem, o_vmem):
                pltpu.sync_copy(t_hbm.at[i_vmem.at[0]], o_vmem)

            pltpu.emit_pipeline(
                body,
                grid=(BT // W,),
                in_specs=[pl.BlockSpec((1, W), lambda i: (0, i))],
                out_specs=[pl.BlockSpec((W, D), lambda i: (i, 0))],
                core_axis_name=("core", "subcore"),
                dimension_semantics=(pltpu.PARALLEL,),
            )(i_hbm, o_hbm)

        gather_into(p_hbm, op_hbm)
        gather_into(e_hbm, oe_hbm)

    return sckern(ptab, etab, pidx2d)


def _mega_kernel(S, T, D, C, NB,
                 x_ref, durc_ref, ptrg_ref, hib_ref, maxd_ref,
                 dw1, db1, dg1, dbe1, dw2, db2, dg2, dbe2, dlw,
                 pw1, pb1, pg1, pbe1, pw2, pb2, pg2, pbe2, plw,
                 ew1, eb1, eg1, ebe1, ew2, eb2, eg2, ebe2, elw,
                 tabs_ref,
                 xe_ref, ppred_ref, epred_ref, dpred_ref, maskf_ref):
    xb = x_ref[0]
    durcol = durc_ref[0]
    ptrg = ptrg_ref[0]
    maxd = maxd_ref[0, 0]

    dpred = _vp_body(xb, dw1, db1[...], dg1[...], dbe1[...],
                     dw2, db2[...], dg2[...], dbe2[...], dlw[...])
    dpred_ref[0, 0, :] = dpred[0]

    r = jax.lax.broadcasted_iota(jnp.int32, (S, S), 0)
    c = jax.lax.broadcasted_iota(jnp.int32, (S, S), 1)
    upper = (r <= c).astype(_BF16)
    cum = jax.lax.dot_general(
        durcol.astype(_BF16), upper, (((0,), (0,)), ((), ())),
        preferred_element_type=_F32)
    durrow = jax.lax.dot_general(
        durcol.astype(_BF16), (r == c).astype(_BF16), (((0,), (0,)), ((), ())),
        preferred_element_type=_F32)
    cum_prev = cum - durrow
    mel_len = cum[0, S - 1]
    lim = jnp.minimum(mel_len, maxd)
    cumc = jnp.minimum(cum, lim)

    tt = jax.lax.broadcasted_iota(jnp.int32, (T, 1), 0).astype(_F32)
    E = ((cum_prev <= tt) & (tt < cumc)).astype(_BF16)
    xe0 = jnp.dot(E, xb.astype(_BF16), preferred_element_type=_F32)
    trow = jax.lax.broadcasted_iota(jnp.int32, (1, T), 1).astype(_F32)
    maskf_ref[0, 0, :] = (trow[0] >= lim).astype(_F32)

    pv = ptrg.reshape(T, 1)
    hib = hib_ref[0:1, :]
    lob = jnp.concatenate([jnp.full((1, 1), -1e30, _F32), hib[:, :NB - 1]],
                          axis=1)
    onehot = ((lob < pv) & (pv <= hib)).astype(_BF16)
    embs = jnp.dot(onehot, tabs_ref[...].astype(_BF16),
                   preferred_element_type=_F32)
    pemb = embs[:, :D]
    eemb = embs[:, D:]

    ppred = _vp_body(xe0, pw1, pb1[...], pg1[...], pbe1[...],
                     pw2, pb2[...], pg2[...], pbe2[...], plw[...])
    ppred_ref[0, 0, :] = ppred[0]
    xe1 = xe0 + pemb

    epred = _vp_body(xe1, ew1, eb1[...], eg1[...], ebe1[...],
                     ew2, eb2[...], eg2[...], ebe2[...], elw[...])
    epred_ref[0, 0, :] = epred[0]
    xe_ref[0] = xe1 + eemb


def _vp_args(p):
    C = p['c1b'].shape[0]
    return (
        p['c1w'].transpose(2, 0, 1), p['c1b'].reshape(1, C),
        p['g1'].reshape(1, C), p['b1'].reshape(1, C),
        p['c2w'].transpose(2, 0, 1), p['c2b'].reshape(1, C),
        p['g2'].reshape(1, C), p['b2'].reshape(1, C),
        p['lw'].reshape(1, C),
    )


def kernel(x, dur_trg, pitch_trg, energy_trg, src_mask, max_dur,
           dp, pp, ep, pitch_bins, energy_bins, pitch_table, energy_table):
    B, S, D = x.shape
    T = pitch_trg.shape[1]
    C = dp['c1b'].shape[0]
    NB = pitch_table.shape[0]
    G = 4

    durc = dur_trg.astype(_F32).reshape(B, S, 1)
    ptrg = pitch_trg.reshape(B, 1, T)
    binsf = pitch_bins.astype(_F32)
    hib = jnp.concatenate([binsf, jnp.full((1,), 1e30, _F32)]).reshape(1, NB)
    hib8 = jnp.broadcast_to(hib, (8, NB))
    binscol = jnp.broadcast_to(hib.reshape(NB, 1), (NB, 128))
    maxd_arr = jnp.full((8, 128), max_dur, _F32)
    tabs = jnp.concatenate([pitch_table, energy_table], axis=1)

    def full(a):
        return pl.BlockSpec(a.shape, lambda b: (0,) * a.ndim)

    row_spec = pl.BlockSpec((1, 1, T), lambda b: (b, 0, 0))

    ptrg_flat = pitch_trg.reshape(B // G, 1, G * T)
    pidx = pl.pallas_call(
        functools.partial(_bidx_kernel, NB),
        grid=(B // G,),
        in_specs=[pl.BlockSpec((1, 1, G * T), lambda b: (b, 0, 0)),
                  full(binscol)],
        out_specs=[pl.BlockSpec((1, 1, G * T), lambda b: (b, 0, 0))],
        out_shape=[jax.ShapeDtypeStruct((B // G, 1, G * T), jnp.int32)],
    )(ptrg_flat, binscol)[0]

    pemb, eemb = _sc_gather(pitch_table, energy_table, pidx.reshape(1, B * T))
    pemb = pemb.reshape(B, T, D)
    eemb = eemb.reshape(B, T, D)

    vp_all = _vp_args(dp) + _vp_args(pp) + _vp_args(ep)
    xe, ppred, epred, dpred, maskf = pl.pallas_call(
        functools.partial(_mega_kernel, S, T, D, C, NB),
        grid=(B,),
        in_specs=[pl.BlockSpec((1, S, D), lambda b: (b, 0, 0)),
                  pl.BlockSpec((1, S, 1), lambda b: (b, 0, 0)),
                  row_spec, full(hib8), full(maxd_arr)]
        + [full(a) for a in vp_all]
        + [full(tabs)],
        out_specs=[pl.BlockSpec((1, T, D), lambda b: (b, 0, 0)),
                   row_spec, row_spec,
                   pl.BlockSpec((1, 1, S), lambda b: (b, 0, 0)),
                   row_spec],
        out_shape=[jax.ShapeDtypeStruct((B, T, D), _F32),
                   jax.ShapeDtypeStruct((B, 1, T), _F32),
                   jax.ShapeDtypeStruct((B, 1, T), _F32),
                   jax.ShapeDtypeStruct((B, 1, S), _F32),
                   jax.ShapeDtypeStruct((B, 1, T), _F32)],
    )(x, durc, ptrg, hib8, maxd_arr, *vp_all, tabs)

    mel_mask = maskf.reshape(B, T) > 0.5
    validf = 1.0 - maskf.reshape(B, T)
    log_dur_pred = jnp.where(src_mask, 0.0, dpred.reshape(B, S) + dp['lb'])
    pitch_pred = (ppred.reshape(B, T) + pp['lb']) * validf
    energy_pred = (epred.reshape(B, T) + ep['lb']) * validf
    return (xe, mel_mask, log_dur_pred, dur_trg,
            pitch_pred, pemb, energy_pred, eemb)
